# baseline (speedup 1.0000x reference)
import functools

import jax
import jax.numpy as jnp
from jax import lax
from jax.experimental import pallas as pl
from jax.experimental.pallas import tpu as pltpu
from jax.experimental.pallas import tpu_sc as plsc

D_MODEL = 768
D_HIDDEN = 1024
NUM_BANKS = 64
TOP_K = 2
T = 2048
NSLOTS = T * TOP_K

TILE_M = 64
ROWS_PAD = 5120

NW = 32
GATHER_CHUNK = 80

_sc_mesh = functools.partial(
    plsc.VectorSubcoreMesh, core_axis_name="c", subcore_axis_name="s"
)


def _selector_kernel(
    x_ref, wsel_ref, bsel_ref,
    p0_ref, p1_ref, pos0_ref, pos1_ref, starts_ref, counts_ref,
):
    x = x_ref[...]
    logits = jnp.dot(x, wsel_ref[...], preferred_element_type=jnp.float32)
    logits = logits + bsel_ref[...]
    m = jnp.max(logits, axis=-1, keepdims=True)
    e = jnp.exp(logits - m)
    probs = e / jnp.sum(e, axis=-1, keepdims=True)

    iota = lax.broadcasted_iota(jnp.int32, probs.shape, 1)
    m0 = jnp.max(probs, axis=-1, keepdims=True)
    i0 = jnp.min(jnp.where(probs == m0, iota, NUM_BANKS), axis=-1, keepdims=True)
    masked = jnp.where(iota == i0, -1.0, probs)
    m1 = jnp.max(masked, axis=-1, keepdims=True)
    i1 = jnp.min(jnp.where(masked == m1, iota, NUM_BANKS), axis=-1, keepdims=True)

    oh0 = (iota == i0).astype(jnp.float32)
    oh1 = (iota == i1).astype(jnp.float32)
    ohsum = oh0 + oh1
    c = ohsum
    d = 1
    while d < T:
        shifted = jnp.concatenate(
            [jnp.zeros((d, NUM_BANKS), c.dtype), c[: T - d, :]], axis=0
        )
        c = c + shifted
        d *= 2
    excl = c - ohsum
    counts = c[T - 1 : T, :]
    counts_i = counts.astype(jnp.int32)
    pad8 = (counts_i + 7) & ~jnp.int32(7)
    s = pad8
    d = 1
    while d < NUM_BANKS:
        shifted = jnp.concatenate(
            [jnp.zeros((1, d), s.dtype), s[:, : NUM_BANKS - d]], axis=1
        )
        s = s + shifted
        d *= 2
    starts = (s - pad8).astype(jnp.float32)

    pos = starts + excl
    pos0 = jnp.sum(oh0 * pos, axis=-1, keepdims=True)
    pos1 = jnp.sum(oh1 * pos, axis=-1, keepdims=True)

    p0_ref[...] = jnp.broadcast_to(m0, (T, 16))
    p1_ref[...] = jnp.broadcast_to(m1, (T, 16))
    pos0_ref[...] = pos0.astype(jnp.int32)
    pos1_ref[...] = pos1.astype(jnp.int32)
    starts_ref[...] = (s - pad8).astype(jnp.int32)
    counts_ref[...] = counts_i


def _selector(x, W_sel, b_sel):
    f32 = jnp.float32
    return pl.pallas_call(
        _selector_kernel,
        out_shape=(
            jax.ShapeDtypeStruct((T, 16), f32),
            jax.ShapeDtypeStruct((T, 16), f32),
            jax.ShapeDtypeStruct((T, 1), jnp.int32),
            jax.ShapeDtypeStruct((T, 1), jnp.int32),
            jax.ShapeDtypeStruct((1, NUM_BANKS), jnp.int32),
            jax.ShapeDtypeStruct((1, NUM_BANKS), jnp.int32),
        ),
    )(x, W_sel, b_sel.reshape(1, NUM_BANKS))


def _sc_scatter_rows_body(src_hbm, pos0_hbm, pos1_hbm, out_hbm, idx_v, rows_v, s0, s1):
    wid = lax.axis_index("s") * 2 + lax.axis_index("c")
    per_w = T // NW
    base = wid * per_w
    pltpu.sync_copy(pos0_hbm.at[pl.ds(base, per_w)], idx_v.at[0])
    pltpu.sync_copy(pos1_hbm.at[pl.ds(base, per_w)], idx_v.at[1])
    pltpu.sync_copy(src_hbm.at[pl.ds(base, per_w)], rows_v)
    c0 = pltpu.async_copy(rows_v, out_hbm.at[idx_v.at[0]], s0)
    c1 = pltpu.async_copy(rows_v, out_hbm.at[idx_v.at[1]], s1)
    c0.wait()
    c1.wait()


def _sc_scatter_rows(src, pos0, pos1):
    per_w = T // NW
    return pl.kernel(
        _sc_scatter_rows_body,
        out_type=jax.ShapeDtypeStruct((ROWS_PAD, D_MODEL), jnp.float32),
        mesh=_sc_mesh(),
        scratch_types=[
            pltpu.VMEM((2, per_w), jnp.int32),
            pltpu.VMEM((per_w, D_MODEL), jnp.float32),
            pltpu.SemaphoreType.DMA,
            pltpu.SemaphoreType.DMA,
        ],
    )(src, pos0, pos1)


def _gmm_kernel(starts_ref, counts_ref, xs_ref, w1_ref, b1_ref, w2_ref, b2_ref, ys_ref):
    e = pl.program_id(0)
    start = pl.multiple_of(starts_ref[e], 8)
    n = counts_ref[e]
    ntiles = (n + TILE_M - 1) // TILE_M
    w1 = w1_ref[0].astype(jnp.bfloat16)
    b1 = b1_ref[0]
    w2 = w2_ref[0].astype(jnp.bfloat16)
    b2 = b2_ref[0]

    def body(j, carry):
        r0 = start + j * TILE_M
        xt = xs_ref[pl.ds(r0, TILE_M), :].astype(jnp.bfloat16)
        h = jnp.dot(xt, w1, preferred_element_type=jnp.float32) + b1
        h = jnp.maximum(h, 0.0)
        yt = jnp.dot(h.astype(jnp.bfloat16), w2, preferred_element_type=jnp.float32) + b2
        ys_ref[pl.ds(r0, TILE_M), :] = yt
        return carry

    lax.fori_loop(0, ntiles, body, 0)


def _gmm(starts, counts, xs, W1, b1, W2, b2):
    return pl.pallas_call(
        _gmm_kernel,
        grid=(NUM_BANKS,),
        in_specs=[
            pl.BlockSpec(memory_space=pltpu.SMEM),
            pl.BlockSpec(memory_space=pltpu.SMEM),
            pl.BlockSpec((ROWS_PAD, D_MODEL), lambda e: (0, 0)),
            pl.BlockSpec((1, D_MODEL, D_HIDDEN), lambda e: (e, 0, 0)),
            pl.BlockSpec((1, 1, D_HIDDEN), lambda e: (e, 0, 0)),
            pl.BlockSpec((1, D_HIDDEN, D_MODEL), lambda e: (e, 0, 0)),
            pl.BlockSpec((1, 1, D_MODEL), lambda e: (e, 0, 0)),
        ],
        out_specs=pl.BlockSpec((ROWS_PAD, D_MODEL), lambda e: (0, 0)),
        out_shape=jax.ShapeDtypeStruct((ROWS_PAD, D_MODEL), jnp.float32),
    )(starts, counts, xs, W1, b1.reshape(NUM_BANKS, 1, D_HIDDEN), W2, b2.reshape(NUM_BANKS, 1, D_MODEL))


def _sc_combine_body(
    ys_hbm, pos0_hbm, pos1_hbm, p0_hbm, p1_hbm, out_hbm,
    i0_v, i1_v, p0_v, p1_v, r0_v, r1_v, s0, s1,
):
    wid = lax.axis_index("s") * 2 + lax.axis_index("c")
    per_w = T // NW
    base = wid * per_w
    pltpu.sync_copy(pos0_hbm.at[pl.ds(base, per_w)], i0_v)
    pltpu.sync_copy(pos1_hbm.at[pl.ds(base, per_w)], i1_v)
    pltpu.sync_copy(p0_hbm.at[pl.ds(base * 16, per_w * 16)], p0_v)
    pltpu.sync_copy(p1_hbm.at[pl.ds(base * 16, per_w * 16)], p1_v)
    c0 = pltpu.async_copy(ys_hbm.at[i0_v], r0_v, s0)
    c1 = pltpu.async_copy(ys_hbm.at[i1_v], r1_v, s1)
    c0.wait()
    c1.wait()

    def row(r, carry):
        pa = p0_v[pl.ds(r * 16, 16)]
        pb = p1_v[pl.ds(r * 16, 16)]
        for c in range(D_MODEL // 16):
            a = r0_v[r, pl.ds(c * 16, 16)]
            b = r1_v[r, pl.ds(c * 16, 16)]
            r0_v[r, pl.ds(c * 16, 16)] = pa * a + pb * b
        return carry

    lax.fori_loop(0, per_w, row, 0)
    pltpu.sync_copy(r0_v, out_hbm.at[pl.ds(base, per_w)])


def _sc_combine(ys, pos0, pos1, p0, p1):
    per_w = T // NW
    f32 = jnp.float32
    return pl.kernel(
        _sc_combine_body,
        out_type=jax.ShapeDtypeStruct((T, D_MODEL), f32),
        mesh=_sc_mesh(),
        scratch_types=[
            pltpu.VMEM((per_w,), jnp.int32),
            pltpu.VMEM((per_w,), jnp.int32),
            pltpu.VMEM((per_w * 16,), f32),
            pltpu.VMEM((per_w * 16,), f32),
            pltpu.VMEM((per_w, D_MODEL), f32),
            pltpu.VMEM((per_w, D_MODEL), f32),
            pltpu.SemaphoreType.DMA,
            pltpu.SemaphoreType.DMA,
        ],
    )(ys, pos0, pos1, p0, p1)


def kernel(tensor, W_sel, b_sel, W1, b1, W2, b2):
    x = tensor.reshape(T, D_MODEL)
    p0, p1, pos0, pos1, starts, counts = _selector(x, W_sel, b_sel)
    pos0 = pos0.reshape(T)
    pos1 = pos1.reshape(T)

    xs = _sc_scatter_rows(x, pos0, pos1)
    ys = _gmm(
        starts.reshape(NUM_BANKS), counts.reshape(NUM_BANKS), xs, W1, b1, W2, b2
    )
    out = _sc_combine(ys, pos0, pos1, p0.reshape(T * 16), p1.reshape(T * 16))
    return out.reshape(tensor.shape)


# speedup vs baseline: 1.0511x; 1.0511x over previous
import functools

import jax
import jax.numpy as jnp
from jax import lax
from jax.experimental import pallas as pl
from jax.experimental.pallas import tpu as pltpu
from jax.experimental.pallas import tpu_sc as plsc

D_MODEL = 768
D_HIDDEN = 1024
NUM_BANKS = 64
TOP_K = 2
T = 2048
NSLOTS = T * TOP_K

TILE_M = 128
ROWS_PAD = 5120

NW = 32
GATHER_CHUNK = 80

_sc_mesh = functools.partial(
    plsc.VectorSubcoreMesh, core_axis_name="c", subcore_axis_name="s"
)


def _selector_kernel(
    x_ref, wsel_ref, bsel_ref,
    p0_ref, p1_ref, pos0_ref, pos1_ref, starts_ref, counts_ref,
):
    x = x_ref[...]
    logits = jnp.dot(x, wsel_ref[...], preferred_element_type=jnp.float32)
    logits = logits + bsel_ref[...]
    m = jnp.max(logits, axis=-1, keepdims=True)
    e = jnp.exp(logits - m)
    probs = e / jnp.sum(e, axis=-1, keepdims=True)

    iota = lax.broadcasted_iota(jnp.int32, probs.shape, 1)
    m0 = jnp.max(probs, axis=-1, keepdims=True)
    i0 = jnp.min(jnp.where(probs == m0, iota, NUM_BANKS), axis=-1, keepdims=True)
    masked = jnp.where(iota == i0, -1.0, probs)
    m1 = jnp.max(masked, axis=-1, keepdims=True)
    i1 = jnp.min(jnp.where(masked == m1, iota, NUM_BANKS), axis=-1, keepdims=True)

    oh0 = (iota == i0).astype(jnp.float32)
    oh1 = (iota == i1).astype(jnp.float32)
    ohsum = oh0 + oh1
    c = ohsum
    d = 1
    while d < T:
        shifted = jnp.concatenate(
            [jnp.zeros((d, NUM_BANKS), c.dtype), c[: T - d, :]], axis=0
        )
        c = c + shifted
        d *= 2
    excl = c - ohsum
    counts = c[T - 1 : T, :]
    counts_i = counts.astype(jnp.int32)
    pad8 = (counts_i + 7) & ~jnp.int32(7)
    s = pad8
    d = 1
    while d < NUM_BANKS:
        shifted = jnp.concatenate(
            [jnp.zeros((1, d), s.dtype), s[:, : NUM_BANKS - d]], axis=1
        )
        s = s + shifted
        d *= 2
    starts = (s - pad8).astype(jnp.float32)

    pos = starts + excl
    pos0 = jnp.sum(oh0 * pos, axis=-1, keepdims=True)
    pos1 = jnp.sum(oh1 * pos, axis=-1, keepdims=True)

    p0_ref[...] = jnp.broadcast_to(m0, (T, 16))
    p1_ref[...] = jnp.broadcast_to(m1, (T, 16))
    pos0_ref[...] = pos0.astype(jnp.int32)
    pos1_ref[...] = pos1.astype(jnp.int32)
    starts_ref[...] = (s - pad8).astype(jnp.int32)
    counts_ref[...] = counts_i


def _selector(x, W_sel, b_sel):
    f32 = jnp.float32
    return pl.pallas_call(
        _selector_kernel,
        out_shape=(
            jax.ShapeDtypeStruct((T, 16), f32),
            jax.ShapeDtypeStruct((T, 16), f32),
            jax.ShapeDtypeStruct((T, 1), jnp.int32),
            jax.ShapeDtypeStruct((T, 1), jnp.int32),
            jax.ShapeDtypeStruct((1, NUM_BANKS), jnp.int32),
            jax.ShapeDtypeStruct((1, NUM_BANKS), jnp.int32),
        ),
    )(x, W_sel, b_sel.reshape(1, NUM_BANKS))


def _sc_scatter_rows_body(src_hbm, pos0_hbm, pos1_hbm, out_hbm, idx_v, rows_v, s0, s1):
    wid = lax.axis_index("s") * 2 + lax.axis_index("c")
    per_w = T // NW
    base = wid * per_w
    pltpu.sync_copy(pos0_hbm.at[pl.ds(base, per_w)], idx_v.at[0])
    pltpu.sync_copy(pos1_hbm.at[pl.ds(base, per_w)], idx_v.at[1])
    pltpu.sync_copy(src_hbm.at[pl.ds(base, per_w)], rows_v)
    c0 = pltpu.async_copy(rows_v, out_hbm.at[idx_v.at[0]], s0)
    c1 = pltpu.async_copy(rows_v, out_hbm.at[idx_v.at[1]], s1)
    c0.wait()
    c1.wait()


def _sc_scatter_rows(src, pos0, pos1):
    per_w = T // NW
    return pl.kernel(
        _sc_scatter_rows_body,
        out_type=jax.ShapeDtypeStruct((ROWS_PAD, D_MODEL), jnp.float32),
        mesh=_sc_mesh(),
        scratch_types=[
            pltpu.VMEM((2, per_w), jnp.int32),
            pltpu.VMEM((per_w, D_MODEL), jnp.float32),
            pltpu.SemaphoreType.DMA,
            pltpu.SemaphoreType.DMA,
        ],
    )(src, pos0, pos1)


def _gmm_kernel(starts_ref, counts_ref, xs_ref, w1_ref, b1_ref, w2_ref, b2_ref, ys_ref):
    e = pl.program_id(0)
    start = pl.multiple_of(starts_ref[e], 8)
    n = counts_ref[e]
    ntiles = (n + TILE_M - 1) // TILE_M
    w1 = w1_ref[0]
    b1 = b1_ref[0]
    w2 = w2_ref[0]
    b2 = b2_ref[0]

    def body(j, carry):
        r0 = start + j * TILE_M
        xt = xs_ref[pl.ds(r0, TILE_M), :]
        h = jnp.dot(xt, w1, preferred_element_type=jnp.float32) + b1
        h = jnp.maximum(h, 0.0)
        yt = jnp.dot(h, w2, preferred_element_type=jnp.float32) + b2
        ys_ref[pl.ds(r0, TILE_M), :] = yt
        return carry

    lax.fori_loop(0, ntiles, body, 0)


def _gmm(starts, counts, xs, W1, b1, W2, b2):
    return pl.pallas_call(
        _gmm_kernel,
        grid=(NUM_BANKS,),
        in_specs=[
            pl.BlockSpec(memory_space=pltpu.SMEM),
            pl.BlockSpec(memory_space=pltpu.SMEM),
            pl.BlockSpec((ROWS_PAD, D_MODEL), lambda e: (0, 0)),
            pl.BlockSpec((1, D_MODEL, D_HIDDEN), lambda e: (e, 0, 0)),
            pl.BlockSpec((1, 1, D_HIDDEN), lambda e: (e, 0, 0)),
            pl.BlockSpec((1, D_HIDDEN, D_MODEL), lambda e: (e, 0, 0)),
            pl.BlockSpec((1, 1, D_MODEL), lambda e: (e, 0, 0)),
        ],
        out_specs=pl.BlockSpec((ROWS_PAD, D_MODEL), lambda e: (0, 0)),
        out_shape=jax.ShapeDtypeStruct((ROWS_PAD, D_MODEL), jnp.float32),
    )(starts, counts, xs, W1, b1.reshape(NUM_BANKS, 1, D_HIDDEN), W2, b2.reshape(NUM_BANKS, 1, D_MODEL))


def _sc_combine_body(
    ys_hbm, pos0_hbm, pos1_hbm, p0_hbm, p1_hbm, out_hbm,
    i0_v, i1_v, p0_v, p1_v, r0_v, r1_v, s0, s1,
):
    wid = lax.axis_index("s") * 2 + lax.axis_index("c")
    per_w = T // NW
    base = wid * per_w
    pltpu.sync_copy(pos0_hbm.at[pl.ds(base, per_w)], i0_v)
    pltpu.sync_copy(pos1_hbm.at[pl.ds(base, per_w)], i1_v)
    pltpu.sync_copy(p0_hbm.at[pl.ds(base * 16, per_w * 16)], p0_v)
    pltpu.sync_copy(p1_hbm.at[pl.ds(base * 16, per_w * 16)], p1_v)
    c0 = pltpu.async_copy(ys_hbm.at[i0_v], r0_v, s0)
    c1 = pltpu.async_copy(ys_hbm.at[i1_v], r1_v, s1)
    c0.wait()
    c1.wait()

    def row(r, carry):
        pa = p0_v[pl.ds(r * 16, 16)]
        pb = p1_v[pl.ds(r * 16, 16)]
        for c in range(D_MODEL // 16):
            a = r0_v[r, pl.ds(c * 16, 16)]
            b = r1_v[r, pl.ds(c * 16, 16)]
            r0_v[r, pl.ds(c * 16, 16)] = pa * a + pb * b
        return carry

    lax.fori_loop(0, per_w, row, 0)
    pltpu.sync_copy(r0_v, out_hbm.at[pl.ds(base, per_w)])


def _sc_combine(ys, pos0, pos1, p0, p1):
    per_w = T // NW
    f32 = jnp.float32
    return pl.kernel(
        _sc_combine_body,
        out_type=jax.ShapeDtypeStruct((T, D_MODEL), f32),
        mesh=_sc_mesh(),
        scratch_types=[
            pltpu.VMEM((per_w,), jnp.int32),
            pltpu.VMEM((per_w,), jnp.int32),
            pltpu.VMEM((per_w * 16,), f32),
            pltpu.VMEM((per_w * 16,), f32),
            pltpu.VMEM((per_w, D_MODEL), f32),
            pltpu.VMEM((per_w, D_MODEL), f32),
            pltpu.SemaphoreType.DMA,
            pltpu.SemaphoreType.DMA,
        ],
    )(ys, pos0, pos1, p0, p1)


def kernel(tensor, W_sel, b_sel, W1, b1, W2, b2):
    x = tensor.reshape(T, D_MODEL)
    p0, p1, pos0, pos1, starts, counts = _selector(x, W_sel, b_sel)
    pos0 = pos0.reshape(T)
    pos1 = pos1.reshape(T)

    xs = _sc_scatter_rows(x, pos0, pos1)
    ys = _gmm(
        starts.reshape(NUM_BANKS), counts.reshape(NUM_BANKS), xs, W1, b1, W2, b2
    )
    out = _sc_combine(ys, pos0, pos1, p0.reshape(T * 16), p1.reshape(T * 16))
    return out.reshape(tensor.shape)


# speedup vs baseline: 1.0540x; 1.0028x over previous
import functools

import jax
import jax.numpy as jnp
from jax import lax
from jax.experimental import pallas as pl
from jax.experimental.pallas import tpu as pltpu
from jax.experimental.pallas import tpu_sc as plsc

D_MODEL = 768
D_HIDDEN = 1024
NUM_BANKS = 64
TOP_K = 2
T = 2048
NSLOTS = T * TOP_K

TILE_M = 128
ROWS_PAD = 4672

NW = 32

_sc_mesh = functools.partial(
    plsc.VectorSubcoreMesh, core_axis_name="c", subcore_axis_name="s"
)


def _selector_kernel(
    x_ref, wsel_ref, bsel_ref,
    p0_ref, p1_ref, pos0_ref, pos1_ref, starts_ref, counts_ref,
):
    x = x_ref[...]
    logits = jnp.dot(x, wsel_ref[...], preferred_element_type=jnp.float32)
    logits = logits + bsel_ref[...]
    m = jnp.max(logits, axis=-1, keepdims=True)
    e = jnp.exp(logits - m)
    probs = e / jnp.sum(e, axis=-1, keepdims=True)

    iota = lax.broadcasted_iota(jnp.int32, probs.shape, 1)
    m0 = jnp.max(probs, axis=-1, keepdims=True)
    i0 = jnp.min(jnp.where(probs == m0, iota, NUM_BANKS), axis=-1, keepdims=True)
    masked = jnp.where(iota == i0, -1.0, probs)
    m1 = jnp.max(masked, axis=-1, keepdims=True)
    i1 = jnp.min(jnp.where(masked == m1, iota, NUM_BANKS), axis=-1, keepdims=True)

    oh0 = (iota == i0).astype(jnp.float32)
    oh1 = (iota == i1).astype(jnp.float32)
    ohsum = oh0 + oh1
    c = ohsum
    d = 1
    while d < T:
        shifted = jnp.concatenate(
            [jnp.zeros((d, NUM_BANKS), c.dtype), c[: T - d, :]], axis=0
        )
        c = c + shifted
        d *= 2
    excl = c - ohsum
    counts = c[T - 1 : T, :]
    counts_i = counts.astype(jnp.int32)
    pad8 = (counts_i + 7) & ~jnp.int32(7)
    s = pad8
    d = 1
    while d < NUM_BANKS:
        shifted = jnp.concatenate(
            [jnp.zeros((1, d), s.dtype), s[:, : NUM_BANKS - d]], axis=1
        )
        s = s + shifted
        d *= 2
    starts = (s - pad8).astype(jnp.float32)

    pos = starts + excl
    pos0 = jnp.sum(oh0 * pos, axis=-1, keepdims=True)
    pos1 = jnp.sum(oh1 * pos, axis=-1, keepdims=True)

    p0_ref[...] = jnp.broadcast_to(m0, (T, 16))
    p1_ref[...] = jnp.broadcast_to(m1, (T, 16))
    pos0_ref[...] = pos0.astype(jnp.int32)
    pos1_ref[...] = pos1.astype(jnp.int32)
    starts_ref[...] = (s - pad8).astype(jnp.int32)
    counts_ref[...] = counts_i


def _selector(x, W_sel, b_sel):
    f32 = jnp.float32
    return pl.pallas_call(
        _selector_kernel,
        out_shape=(
            jax.ShapeDtypeStruct((T, 16), f32),
            jax.ShapeDtypeStruct((T, 16), f32),
            jax.ShapeDtypeStruct((T, 1), jnp.int32),
            jax.ShapeDtypeStruct((T, 1), jnp.int32),
            jax.ShapeDtypeStruct((1, NUM_BANKS), jnp.int32),
            jax.ShapeDtypeStruct((1, NUM_BANKS), jnp.int32),
        ),
    )(x, W_sel, b_sel.reshape(1, NUM_BANKS))


def _sc_scatter_rows_body(src_hbm, pos0_hbm, pos1_hbm, out_hbm, idx_v, rows_v, s0, s1):
    wid = lax.axis_index("s") * 2 + lax.axis_index("c")
    per_w = T // NW
    base = wid * per_w
    pltpu.sync_copy(pos0_hbm.at[pl.ds(base, per_w)], idx_v.at[0])
    pltpu.sync_copy(pos1_hbm.at[pl.ds(base, per_w)], idx_v.at[1])
    pltpu.sync_copy(src_hbm.at[pl.ds(base, per_w)], rows_v)
    c0 = pltpu.async_copy(rows_v, out_hbm.at[idx_v.at[0]], s0)
    c1 = pltpu.async_copy(rows_v, out_hbm.at[idx_v.at[1]], s1)
    c0.wait()
    c1.wait()


def _sc_scatter_rows(src, pos0, pos1):
    per_w = T // NW
    return pl.kernel(
        _sc_scatter_rows_body,
        out_type=jax.ShapeDtypeStruct((ROWS_PAD, D_MODEL), jnp.float32),
        mesh=_sc_mesh(),
        scratch_types=[
            pltpu.VMEM((2, per_w), jnp.int32),
            pltpu.VMEM((per_w, D_MODEL), jnp.float32),
            pltpu.SemaphoreType.DMA,
            pltpu.SemaphoreType.DMA,
        ],
    )(src, pos0, pos1)


def _gmm_kernel(starts_ref, counts_ref, xs_ref, w1_ref, b1_ref, w2_ref, b2_ref, ys_ref):
    e = pl.program_id(0)
    start = pl.multiple_of(starts_ref[e], 8)
    n = counts_ref[e]
    ntiles = (n + TILE_M - 1) // TILE_M
    w1 = w1_ref[0]
    b1 = b1_ref[0]
    w2 = w2_ref[0]
    b2 = b2_ref[0]

    def body(j, carry):
        r0 = start + j * TILE_M
        xt = xs_ref[pl.ds(r0, TILE_M), :]
        h = jnp.dot(xt, w1, preferred_element_type=jnp.float32) + b1
        h = jnp.maximum(h, 0.0)
        yt = jnp.dot(h, w2, preferred_element_type=jnp.float32) + b2
        ys_ref[pl.ds(r0, TILE_M), :] = yt
        return carry

    lax.fori_loop(0, ntiles, body, 0)


def _gmm(starts, counts, xs, W1, b1, W2, b2):
    return pl.pallas_call(
        _gmm_kernel,
        grid=(NUM_BANKS,),
        in_specs=[
            pl.BlockSpec(memory_space=pltpu.SMEM),
            pl.BlockSpec(memory_space=pltpu.SMEM),
            pl.BlockSpec((ROWS_PAD, D_MODEL), lambda e: (0, 0)),
            pl.BlockSpec((1, D_MODEL, D_HIDDEN), lambda e: (e, 0, 0)),
            pl.BlockSpec((1, 1, D_HIDDEN), lambda e: (e, 0, 0)),
            pl.BlockSpec((1, D_HIDDEN, D_MODEL), lambda e: (e, 0, 0)),
            pl.BlockSpec((1, 1, D_MODEL), lambda e: (e, 0, 0)),
        ],
        out_specs=pl.BlockSpec((ROWS_PAD, D_MODEL), lambda e: (0, 0)),
        out_shape=jax.ShapeDtypeStruct((ROWS_PAD, D_MODEL), jnp.float32),
    )(starts, counts, xs, W1, b1.reshape(NUM_BANKS, 1, D_HIDDEN), W2, b2.reshape(NUM_BANKS, 1, D_MODEL))


def _sc_combine_body(
    ys_hbm, pos0_hbm, pos1_hbm, p0_hbm, p1_hbm, out_hbm,
    i0_v, i1_v, p0_v, p1_v, r0_v, r1_v, s0, s1,
):
    wid = lax.axis_index("s") * 2 + lax.axis_index("c")
    per_w = T // NW
    base = wid * per_w
    pltpu.sync_copy(pos0_hbm.at[pl.ds(base, per_w)], i0_v)
    pltpu.sync_copy(pos1_hbm.at[pl.ds(base, per_w)], i1_v)
    pltpu.sync_copy(p0_hbm.at[pl.ds(base * 16, per_w * 16)], p0_v)
    pltpu.sync_copy(p1_hbm.at[pl.ds(base * 16, per_w * 16)], p1_v)
    c0 = pltpu.async_copy(ys_hbm.at[i0_v], r0_v, s0)
    c1 = pltpu.async_copy(ys_hbm.at[i1_v], r1_v, s1)
    c0.wait()
    c1.wait()

    def row(r, carry):
        pa = p0_v[pl.ds(r * 16, 16)]
        pb = p1_v[pl.ds(r * 16, 16)]
        for c in range(D_MODEL // 16):
            a = r0_v[r, pl.ds(c * 16, 16)]
            b = r1_v[r, pl.ds(c * 16, 16)]
            r0_v[r, pl.ds(c * 16, 16)] = pa * a + pb * b
        return carry

    lax.fori_loop(0, per_w, row, 0)
    pltpu.sync_copy(r0_v, out_hbm.at[pl.ds(base, per_w)])


def _sc_combine(ys, pos0, pos1, p0, p1):
    per_w = T // NW
    f32 = jnp.float32
    return pl.kernel(
        _sc_combine_body,
        out_type=jax.ShapeDtypeStruct((T, D_MODEL), f32),
        mesh=_sc_mesh(),
        scratch_types=[
            pltpu.VMEM((per_w,), jnp.int32),
            pltpu.VMEM((per_w,), jnp.int32),
            pltpu.VMEM((per_w * 16,), f32),
            pltpu.VMEM((per_w * 16,), f32),
            pltpu.VMEM((per_w, D_MODEL), f32),
            pltpu.VMEM((per_w, D_MODEL), f32),
            pltpu.SemaphoreType.DMA,
            pltpu.SemaphoreType.DMA,
        ],
    )(ys, pos0, pos1, p0, p1)


def kernel(tensor, W_sel, b_sel, W1, b1, W2, b2):
    x = tensor.reshape(T, D_MODEL)
    p0, p1, pos0, pos1, starts, counts = _selector(x, W_sel, b_sel)
    pos0 = pos0.reshape(T)
    pos1 = pos1.reshape(T)

    xs = _sc_scatter_rows(x, pos0, pos1)
    ys = _gmm(
        starts.reshape(NUM_BANKS), counts.reshape(NUM_BANKS), xs, W1, b1, W2, b2
    )
    out = _sc_combine(ys, pos0, pos1, p0.reshape(T * 16), p1.reshape(T * 16))
    return out.reshape(tensor.shape)


# speedup vs baseline: 1.1727x; 1.1126x over previous
import functools

import jax
import jax.numpy as jnp
from jax import lax
from jax.experimental import pallas as pl
from jax.experimental.pallas import tpu as pltpu
from jax.experimental.pallas import tpu_sc as plsc

D_MODEL = 768
D_HIDDEN = 1024
NUM_BANKS = 64
TOP_K = 2
T = 2048
NSLOTS = T * TOP_K

TILE_M = 128
ROWS_PAD = 4672

NW = 32

_sc_mesh = functools.partial(
    plsc.VectorSubcoreMesh, core_axis_name="c", subcore_axis_name="s"
)


def _selector_kernel(
    x_ref, wsel_ref, bsel_ref,
    p0_ref, p1_ref, pos0_ref, pos1_ref, starts_ref, counts_ref,
):
    x = x_ref[...]
    logits = jnp.dot(x, wsel_ref[...], preferred_element_type=jnp.float32)
    logits = logits + bsel_ref[...]
    m = jnp.max(logits, axis=-1, keepdims=True)
    e = jnp.exp(logits - m)
    probs = e / jnp.sum(e, axis=-1, keepdims=True)

    iota = lax.broadcasted_iota(jnp.int32, probs.shape, 1)
    m0 = jnp.max(probs, axis=-1, keepdims=True)
    i0 = jnp.min(jnp.where(probs == m0, iota, NUM_BANKS), axis=-1, keepdims=True)
    masked = jnp.where(iota == i0, -1.0, probs)
    m1 = jnp.max(masked, axis=-1, keepdims=True)
    i1 = jnp.min(jnp.where(masked == m1, iota, NUM_BANKS), axis=-1, keepdims=True)

    oh0 = (iota == i0).astype(jnp.float32)
    oh1 = (iota == i1).astype(jnp.float32)
    ohsum = oh0 + oh1
    c = ohsum
    d = 1
    while d < T:
        shifted = jnp.concatenate(
            [jnp.zeros((d, NUM_BANKS), c.dtype), c[: T - d, :]], axis=0
        )
        c = c + shifted
        d *= 2
    excl = c - ohsum
    counts = c[T - 1 : T, :]
    counts_i = counts.astype(jnp.int32)
    pad8 = (counts_i + 7) & ~jnp.int32(7)
    s = pad8
    d = 1
    while d < NUM_BANKS:
        shifted = jnp.concatenate(
            [jnp.zeros((1, d), s.dtype), s[:, : NUM_BANKS - d]], axis=1
        )
        s = s + shifted
        d *= 2
    starts = (s - pad8).astype(jnp.float32)

    pos = starts + excl
    pos0 = jnp.sum(oh0 * pos, axis=-1, keepdims=True)
    pos1 = jnp.sum(oh1 * pos, axis=-1, keepdims=True)

    p0_ref[...] = jnp.broadcast_to(m0, (T, 16))
    p1_ref[...] = jnp.broadcast_to(m1, (T, 16))
    pos0_ref[...] = pos0.astype(jnp.int32)
    pos1_ref[...] = pos1.astype(jnp.int32)
    starts_ref[...] = (s - pad8).astype(jnp.int32)
    counts_ref[...] = counts_i


def _selector(x, W_sel, b_sel):
    f32 = jnp.float32
    return pl.pallas_call(
        _selector_kernel,
        out_shape=(
            jax.ShapeDtypeStruct((T, 16), f32),
            jax.ShapeDtypeStruct((T, 16), f32),
            jax.ShapeDtypeStruct((T, 1), jnp.int32),
            jax.ShapeDtypeStruct((T, 1), jnp.int32),
            jax.ShapeDtypeStruct((1, NUM_BANKS), jnp.int32),
            jax.ShapeDtypeStruct((1, NUM_BANKS), jnp.int32),
        ),
    )(x, W_sel, b_sel.reshape(1, NUM_BANKS))


def _sc_scatter_rows_body(src_hbm, pos0_hbm, pos1_hbm, out_hbm, idx_v, rows_v, s0, s1):
    wid = lax.axis_index("s") * 2 + lax.axis_index("c")
    per_w = T // NW
    base = wid * per_w
    pltpu.sync_copy(pos0_hbm.at[pl.ds(base, per_w)], idx_v.at[0])
    pltpu.sync_copy(pos1_hbm.at[pl.ds(base, per_w)], idx_v.at[1])
    pltpu.sync_copy(src_hbm.at[pl.ds(base, per_w)], rows_v)
    c0 = pltpu.async_copy(rows_v, out_hbm.at[idx_v.at[0]], s0)
    c1 = pltpu.async_copy(rows_v, out_hbm.at[idx_v.at[1]], s1)
    c0.wait()
    c1.wait()


def _sc_scatter_rows(src, pos0, pos1):
    per_w = T // NW
    return pl.kernel(
        _sc_scatter_rows_body,
        out_type=jax.ShapeDtypeStruct((ROWS_PAD, D_MODEL), jnp.float32),
        mesh=_sc_mesh(),
        scratch_types=[
            pltpu.VMEM((2, per_w), jnp.int32),
            pltpu.VMEM((per_w, D_MODEL), jnp.float32),
            pltpu.SemaphoreType.DMA,
            pltpu.SemaphoreType.DMA,
        ],
    )(src, pos0, pos1)


NBUF = 3


def _gmm_kernel(
    starts_ref, counts_ref, xs_ref, w1_hbm, b1_ref, w2_hbm, b2_ref, ys_ref,
    w1b, w2b, sem1, sem2,
):
    e = pl.program_id(0)

    @pl.when(e == 0)
    def _():
        for s in range(2):
            pltpu.make_async_copy(w1_hbm.at[s], w1b.at[s], sem1.at[s]).start()
            pltpu.make_async_copy(w2_hbm.at[s], w2b.at[s], sem2.at[s]).start()

    slot = lax.rem(e, NBUF)
    pltpu.make_async_copy(w1_hbm.at[e], w1b.at[slot], sem1.at[slot]).wait()
    pltpu.make_async_copy(w2_hbm.at[e], w2b.at[slot], sem2.at[slot]).wait()

    @pl.when(e + 2 < NUM_BANKS)
    def _():
        nslot = lax.rem(e + 2, NBUF)
        pltpu.make_async_copy(w1_hbm.at[e + 2], w1b.at[nslot], sem1.at[nslot]).start()
        pltpu.make_async_copy(w2_hbm.at[e + 2], w2b.at[nslot], sem2.at[nslot]).start()

    start = pl.multiple_of(starts_ref[e], 8)
    n = counts_ref[e]
    ntiles = (n + TILE_M - 1) // TILE_M
    b1 = b1_ref[0]
    b2 = b2_ref[0]

    def body(j, carry):
        r0 = start + j * TILE_M
        xt = xs_ref[pl.ds(r0, TILE_M), :]
        h = jnp.dot(xt, w1b[slot], preferred_element_type=jnp.float32) + b1
        h = jnp.maximum(h, 0.0)
        yt = jnp.dot(h, w2b[slot], preferred_element_type=jnp.float32) + b2
        ys_ref[pl.ds(r0, TILE_M), :] = yt
        return carry

    lax.fori_loop(0, ntiles, body, 0)


def _gmm(starts, counts, xs, W1, b1, W2, b2):
    f32 = jnp.float32
    return pl.pallas_call(
        _gmm_kernel,
        grid=(NUM_BANKS,),
        in_specs=[
            pl.BlockSpec(memory_space=pltpu.SMEM),
            pl.BlockSpec(memory_space=pltpu.SMEM),
            pl.BlockSpec((ROWS_PAD, D_MODEL), lambda e: (0, 0)),
            pl.BlockSpec(memory_space=pl.ANY),
            pl.BlockSpec((1, 1, D_HIDDEN), lambda e: (e, 0, 0)),
            pl.BlockSpec(memory_space=pl.ANY),
            pl.BlockSpec((1, 1, D_MODEL), lambda e: (e, 0, 0)),
        ],
        out_specs=pl.BlockSpec((ROWS_PAD, D_MODEL), lambda e: (0, 0)),
        out_shape=jax.ShapeDtypeStruct((ROWS_PAD, D_MODEL), f32),
        scratch_shapes=[
            pltpu.VMEM((NBUF, D_MODEL, D_HIDDEN), f32),
            pltpu.VMEM((NBUF, D_HIDDEN, D_MODEL), f32),
            pltpu.SemaphoreType.DMA((NBUF,)),
            pltpu.SemaphoreType.DMA((NBUF,)),
        ],
    )(starts, counts, xs, W1, b1.reshape(NUM_BANKS, 1, D_HIDDEN), W2, b2.reshape(NUM_BANKS, 1, D_MODEL))


def _sc_combine_body(
    ys_hbm, pos0_hbm, pos1_hbm, p0_hbm, p1_hbm, out_hbm,
    i0_v, i1_v, p0_v, p1_v, r0_v, r1_v, s0, s1,
):
    wid = lax.axis_index("s") * 2 + lax.axis_index("c")
    per_w = T // NW
    base = wid * per_w
    pltpu.sync_copy(pos0_hbm.at[pl.ds(base, per_w)], i0_v)
    pltpu.sync_copy(pos1_hbm.at[pl.ds(base, per_w)], i1_v)
    pltpu.sync_copy(p0_hbm.at[pl.ds(base * 16, per_w * 16)], p0_v)
    pltpu.sync_copy(p1_hbm.at[pl.ds(base * 16, per_w * 16)], p1_v)
    c0 = pltpu.async_copy(ys_hbm.at[i0_v], r0_v, s0)
    c1 = pltpu.async_copy(ys_hbm.at[i1_v], r1_v, s1)
    c0.wait()
    c1.wait()

    def row(r, carry):
        pa = p0_v[pl.ds(r * 16, 16)]
        pb = p1_v[pl.ds(r * 16, 16)]
        for c in range(D_MODEL // 16):
            a = r0_v[r, pl.ds(c * 16, 16)]
            b = r1_v[r, pl.ds(c * 16, 16)]
            r0_v[r, pl.ds(c * 16, 16)] = pa * a + pb * b
        return carry

    lax.fori_loop(0, per_w, row, 0)
    pltpu.sync_copy(r0_v, out_hbm.at[pl.ds(base, per_w)])


def _sc_combine(ys, pos0, pos1, p0, p1):
    per_w = T // NW
    f32 = jnp.float32
    return pl.kernel(
        _sc_combine_body,
        out_type=jax.ShapeDtypeStruct((T, D_MODEL), f32),
        mesh=_sc_mesh(),
        scratch_types=[
            pltpu.VMEM((per_w,), jnp.int32),
            pltpu.VMEM((per_w,), jnp.int32),
            pltpu.VMEM((per_w * 16,), f32),
            pltpu.VMEM((per_w * 16,), f32),
            pltpu.VMEM((per_w, D_MODEL), f32),
            pltpu.VMEM((per_w, D_MODEL), f32),
            pltpu.SemaphoreType.DMA,
            pltpu.SemaphoreType.DMA,
        ],
    )(ys, pos0, pos1, p0, p1)


def kernel(tensor, W_sel, b_sel, W1, b1, W2, b2):
    x = tensor.reshape(T, D_MODEL)
    p0, p1, pos0, pos1, starts, counts = _selector(x, W_sel, b_sel)
    pos0 = pos0.reshape(T)
    pos1 = pos1.reshape(T)

    xs = _sc_scatter_rows(x, pos0, pos1)
    ys = _gmm(
        starts.reshape(NUM_BANKS), counts.reshape(NUM_BANKS), xs, W1, b1, W2, b2
    )
    out = _sc_combine(ys, pos0, pos1, p0.reshape(T * 16), p1.reshape(T * 16))
    return out.reshape(tensor.shape)


# speedup vs baseline: 1.1805x; 1.0066x over previous
import functools

import jax
import jax.numpy as jnp
from jax import lax
from jax.experimental import pallas as pl
from jax.experimental.pallas import tpu as pltpu
from jax.experimental.pallas import tpu_sc as plsc

D_MODEL = 768
D_HIDDEN = 1024
NUM_BANKS = 64
TOP_K = 2
T = 2048
NSLOTS = T * TOP_K

TILE_M = 128
ROWS_PAD = 4672

NW = 32

_sc_mesh = functools.partial(
    plsc.VectorSubcoreMesh, core_axis_name="c", subcore_axis_name="s"
)


def _selector_kernel(
    x_ref, wsel_ref, bsel_ref,
    p0_ref, p1_ref, pos0_ref, pos1_ref, starts_ref, counts_ref,
):
    x = x_ref[...]
    logits = jnp.dot(x, wsel_ref[...], preferred_element_type=jnp.float32)
    logits = logits + bsel_ref[...]
    m = jnp.max(logits, axis=-1, keepdims=True)
    e = jnp.exp(logits - m)
    probs = e / jnp.sum(e, axis=-1, keepdims=True)

    iota = lax.broadcasted_iota(jnp.int32, probs.shape, 1)
    m0 = jnp.max(probs, axis=-1, keepdims=True)
    i0 = jnp.min(jnp.where(probs == m0, iota, NUM_BANKS), axis=-1, keepdims=True)
    masked = jnp.where(iota == i0, -1.0, probs)
    m1 = jnp.max(masked, axis=-1, keepdims=True)
    i1 = jnp.min(jnp.where(masked == m1, iota, NUM_BANKS), axis=-1, keepdims=True)

    oh0 = (iota == i0).astype(jnp.float32)
    oh1 = (iota == i1).astype(jnp.float32)
    ohsum = oh0 + oh1
    c = ohsum
    d = 1
    while d < T:
        shifted = jnp.concatenate(
            [jnp.zeros((d, NUM_BANKS), c.dtype), c[: T - d, :]], axis=0
        )
        c = c + shifted
        d *= 2
    excl = c - ohsum
    counts = c[T - 1 : T, :]
    counts_i = counts.astype(jnp.int32)
    pad8 = (counts_i + 7) & ~jnp.int32(7)
    s = pad8
    d = 1
    while d < NUM_BANKS:
        shifted = jnp.concatenate(
            [jnp.zeros((1, d), s.dtype), s[:, : NUM_BANKS - d]], axis=1
        )
        s = s + shifted
        d *= 2
    starts = (s - pad8).astype(jnp.float32)

    pos = starts + excl
    pos0 = jnp.sum(oh0 * pos, axis=-1, keepdims=True)
    pos1 = jnp.sum(oh1 * pos, axis=-1, keepdims=True)

    p0_ref[...] = jnp.broadcast_to(m0, (T, 16))
    p1_ref[...] = jnp.broadcast_to(m1, (T, 16))
    pos0_ref[...] = pos0.astype(jnp.int32)
    pos1_ref[...] = pos1.astype(jnp.int32)
    starts_ref[...] = (s - pad8).astype(jnp.int32)
    counts_ref[...] = counts_i


def _selector(x, W_sel, b_sel):
    f32 = jnp.float32
    return pl.pallas_call(
        _selector_kernel,
        out_shape=(
            jax.ShapeDtypeStruct((T, 16), f32),
            jax.ShapeDtypeStruct((T, 16), f32),
            jax.ShapeDtypeStruct((T, 1), jnp.int32),
            jax.ShapeDtypeStruct((T, 1), jnp.int32),
            jax.ShapeDtypeStruct((1, NUM_BANKS), jnp.int32),
            jax.ShapeDtypeStruct((1, NUM_BANKS), jnp.int32),
        ),
    )(x, W_sel, b_sel.reshape(1, NUM_BANKS))


def _sc_scatter_rows_body(src_hbm, pos0_hbm, pos1_hbm, out_hbm, idx_v, rows_v, s0, s1):
    wid = lax.axis_index("s") * 2 + lax.axis_index("c")
    per_w = T // NW
    base = wid * per_w
    h0 = pltpu.async_copy(pos0_hbm.at[pl.ds(base, per_w)], idx_v.at[0], s0)
    h1 = pltpu.async_copy(pos1_hbm.at[pl.ds(base, per_w)], idx_v.at[1], s0)
    h2 = pltpu.async_copy(src_hbm.at[pl.ds(base, per_w)], rows_v, s1)
    h0.wait()
    h1.wait()
    h2.wait()
    c0 = pltpu.async_copy(rows_v, out_hbm.at[idx_v.at[0]], s0)
    c1 = pltpu.async_copy(rows_v, out_hbm.at[idx_v.at[1]], s1)
    c0.wait()
    c1.wait()


def _sc_scatter_rows(src, pos0, pos1):
    per_w = T // NW
    return pl.kernel(
        _sc_scatter_rows_body,
        out_type=jax.ShapeDtypeStruct((ROWS_PAD, D_MODEL), jnp.float32),
        mesh=_sc_mesh(),
        scratch_types=[
            pltpu.VMEM((2, per_w), jnp.int32),
            pltpu.VMEM((per_w, D_MODEL), jnp.float32),
            pltpu.SemaphoreType.DMA,
            pltpu.SemaphoreType.DMA,
        ],
    )(src, pos0, pos1)


NBUF = 3


def _gmm_kernel(
    starts_ref, counts_ref, xs_hbm, w1_hbm, b1_ref, w2_hbm, b2_ref, ys_ref,
    xs_ref, w1b, w2b, semx, sem1, sem2,
):
    e = pl.program_id(0)

    @pl.when(e == 0)
    def _():
        pltpu.make_async_copy(xs_hbm, xs_ref, semx).start()
        for s in range(2):
            pltpu.make_async_copy(w1_hbm.at[s], w1b.at[s], sem1.at[s]).start()
            pltpu.make_async_copy(w2_hbm.at[s], w2b.at[s], sem2.at[s]).start()
        pltpu.make_async_copy(xs_hbm, xs_ref, semx).wait()

    slot = lax.rem(e, NBUF)
    pltpu.make_async_copy(w1_hbm.at[e], w1b.at[slot], sem1.at[slot]).wait()
    pltpu.make_async_copy(w2_hbm.at[e], w2b.at[slot], sem2.at[slot]).wait()

    @pl.when(e + 2 < NUM_BANKS)
    def _():
        nslot = lax.rem(e + 2, NBUF)
        pltpu.make_async_copy(w1_hbm.at[e + 2], w1b.at[nslot], sem1.at[nslot]).start()
        pltpu.make_async_copy(w2_hbm.at[e + 2], w2b.at[nslot], sem2.at[nslot]).start()

    start = pl.multiple_of(starts_ref[e], 8)
    n = counts_ref[e]
    ntiles = (n + TILE_M - 1) // TILE_M
    b1 = b1_ref[0]
    b2 = b2_ref[0]

    def body(j, carry):
        r0 = start + j * TILE_M
        xt = xs_ref[pl.ds(r0, TILE_M), :]
        h = jnp.dot(xt, w1b[slot], preferred_element_type=jnp.float32) + b1
        h = jnp.maximum(h, 0.0)
        yt = jnp.dot(h, w2b[slot], preferred_element_type=jnp.float32) + b2
        ys_ref[pl.ds(r0, TILE_M), :] = yt
        return carry

    lax.fori_loop(0, ntiles, body, 0)


def _gmm(starts, counts, xs, W1, b1, W2, b2):
    f32 = jnp.float32
    return pl.pallas_call(
        _gmm_kernel,
        grid=(NUM_BANKS,),
        in_specs=[
            pl.BlockSpec(memory_space=pltpu.SMEM),
            pl.BlockSpec(memory_space=pltpu.SMEM),
            pl.BlockSpec(memory_space=pl.ANY),
            pl.BlockSpec(memory_space=pl.ANY),
            pl.BlockSpec((1, 1, D_HIDDEN), lambda e: (e, 0, 0)),
            pl.BlockSpec(memory_space=pl.ANY),
            pl.BlockSpec((1, 1, D_MODEL), lambda e: (e, 0, 0)),
        ],
        out_specs=pl.BlockSpec((ROWS_PAD, D_MODEL), lambda e: (0, 0)),
        out_shape=jax.ShapeDtypeStruct((ROWS_PAD, D_MODEL), f32),
        scratch_shapes=[
            pltpu.VMEM((ROWS_PAD, D_MODEL), f32),
            pltpu.VMEM((NBUF, D_MODEL, D_HIDDEN), f32),
            pltpu.VMEM((NBUF, D_HIDDEN, D_MODEL), f32),
            pltpu.SemaphoreType.DMA,
            pltpu.SemaphoreType.DMA((NBUF,)),
            pltpu.SemaphoreType.DMA((NBUF,)),
        ],
    )(starts, counts, xs, W1, b1.reshape(NUM_BANKS, 1, D_HIDDEN), W2, b2.reshape(NUM_BANKS, 1, D_MODEL))


def _sc_combine_body(
    ys_hbm, pos0_hbm, pos1_hbm, p0_hbm, p1_hbm, out_hbm,
    i0_v, i1_v, p0_v, p1_v, r0_v, r1_v, s0, s1,
):
    wid = lax.axis_index("s") * 2 + lax.axis_index("c")
    per_w = T // NW
    base = wid * per_w
    h0 = pltpu.async_copy(pos0_hbm.at[pl.ds(base, per_w)], i0_v, s0)
    h1 = pltpu.async_copy(pos1_hbm.at[pl.ds(base, per_w)], i1_v, s0)
    h2 = pltpu.async_copy(p0_hbm.at[pl.ds(base * 16, per_w * 16)], p0_v, s1)
    h3 = pltpu.async_copy(p1_hbm.at[pl.ds(base * 16, per_w * 16)], p1_v, s1)
    h0.wait()
    h1.wait()
    h2.wait()
    h3.wait()
    c0 = pltpu.async_copy(ys_hbm.at[i0_v], r0_v, s0)
    c1 = pltpu.async_copy(ys_hbm.at[i1_v], r1_v, s1)
    c0.wait()
    c1.wait()

    def row(r, carry):
        pa = p0_v[pl.ds(r * 16, 16)]
        pb = p1_v[pl.ds(r * 16, 16)]
        for c in range(D_MODEL // 16):
            a = r0_v[r, pl.ds(c * 16, 16)]
            b = r1_v[r, pl.ds(c * 16, 16)]
            r0_v[r, pl.ds(c * 16, 16)] = pa * a + pb * b
        return carry

    lax.fori_loop(0, per_w, row, 0)
    pltpu.sync_copy(r0_v, out_hbm.at[pl.ds(base, per_w)])


def _sc_combine(ys, pos0, pos1, p0, p1):
    per_w = T // NW
    f32 = jnp.float32
    return pl.kernel(
        _sc_combine_body,
        out_type=jax.ShapeDtypeStruct((T, D_MODEL), f32),
        mesh=_sc_mesh(),
        scratch_types=[
            pltpu.VMEM((per_w,), jnp.int32),
            pltpu.VMEM((per_w,), jnp.int32),
            pltpu.VMEM((per_w * 16,), f32),
            pltpu.VMEM((per_w * 16,), f32),
            pltpu.VMEM((per_w, D_MODEL), f32),
            pltpu.VMEM((per_w, D_MODEL), f32),
            pltpu.SemaphoreType.DMA,
            pltpu.SemaphoreType.DMA,
        ],
    )(ys, pos0, pos1, p0, p1)


def kernel(tensor, W_sel, b_sel, W1, b1, W2, b2):
    x = tensor.reshape(T, D_MODEL)
    p0, p1, pos0, pos1, starts, counts = _selector(x, W_sel, b_sel)
    pos0 = pos0.reshape(T)
    pos1 = pos1.reshape(T)

    xs = _sc_scatter_rows(x, pos0, pos1)
    ys = _gmm(
        starts.reshape(NUM_BANKS), counts.reshape(NUM_BANKS), xs, W1, b1, W2, b2
    )
    out = _sc_combine(ys, pos0, pos1, p0.reshape(T * 16), p1.reshape(T * 16))
    return out.reshape(tensor.shape)


# speedup vs baseline: 1.1814x; 1.0007x over previous
import functools

import jax
import jax.numpy as jnp
from jax import lax
from jax.experimental import pallas as pl
from jax.experimental.pallas import tpu as pltpu
from jax.experimental.pallas import tpu_sc as plsc

D_MODEL = 768
D_HIDDEN = 1024
NUM_BANKS = 64
TOP_K = 2
T = 2048
NSLOTS = T * TOP_K

TILE_M = 128
ROWS_PAD = 4672

NW = 32

_sc_mesh = functools.partial(
    plsc.VectorSubcoreMesh, core_axis_name="c", subcore_axis_name="s"
)


def _selector_kernel(
    x_ref, wsel_ref, bsel_ref,
    p0_ref, p1_ref, pos0_ref, pos1_ref, starts_ref, counts_ref,
):
    x = x_ref[...]
    logits = jnp.dot(x, wsel_ref[...], preferred_element_type=jnp.float32)
    logits = logits + bsel_ref[...]
    m = jnp.max(logits, axis=-1, keepdims=True)
    e = jnp.exp(logits - m)
    probs = e / jnp.sum(e, axis=-1, keepdims=True)

    iota = lax.broadcasted_iota(jnp.int32, probs.shape, 1)
    m0 = jnp.max(probs, axis=-1, keepdims=True)
    i0 = jnp.min(jnp.where(probs == m0, iota, NUM_BANKS), axis=-1, keepdims=True)
    masked = jnp.where(iota == i0, -1.0, probs)
    m1 = jnp.max(masked, axis=-1, keepdims=True)
    i1 = jnp.min(jnp.where(masked == m1, iota, NUM_BANKS), axis=-1, keepdims=True)

    oh0 = (iota == i0).astype(jnp.float32)
    oh1 = (iota == i1).astype(jnp.float32)
    ohsum = oh0 + oh1
    c = ohsum
    d = 1
    while d < T:
        shifted = jnp.concatenate(
            [jnp.zeros((d, NUM_BANKS), c.dtype), c[: T - d, :]], axis=0
        )
        c = c + shifted
        d *= 2
    excl = c - ohsum
    counts = c[T - 1 : T, :]
    counts_i = counts.astype(jnp.int32)
    pad8 = (counts_i + 7) & ~jnp.int32(7)
    s = pad8
    d = 1
    while d < NUM_BANKS:
        shifted = jnp.concatenate(
            [jnp.zeros((1, d), s.dtype), s[:, : NUM_BANKS - d]], axis=1
        )
        s = s + shifted
        d *= 2
    starts = (s - pad8).astype(jnp.float32)

    pos = starts + excl
    pos0 = jnp.sum(oh0 * pos, axis=-1, keepdims=True)
    pos1 = jnp.sum(oh1 * pos, axis=-1, keepdims=True)

    p0_ref[...] = jnp.broadcast_to(m0, (T, 16))
    p1_ref[...] = jnp.broadcast_to(m1, (T, 16))
    pos0_ref[...] = pos0.astype(jnp.int32)
    pos1_ref[...] = pos1.astype(jnp.int32)
    starts_ref[...] = (s - pad8).astype(jnp.int32)
    counts_ref[...] = counts_i


def _selector(x, W_sel, b_sel):
    f32 = jnp.float32
    return pl.pallas_call(
        _selector_kernel,
        out_shape=(
            jax.ShapeDtypeStruct((T, 16), f32),
            jax.ShapeDtypeStruct((T, 16), f32),
            jax.ShapeDtypeStruct((T, 1), jnp.int32),
            jax.ShapeDtypeStruct((T, 1), jnp.int32),
            jax.ShapeDtypeStruct((1, NUM_BANKS), jnp.int32),
            jax.ShapeDtypeStruct((1, NUM_BANKS), jnp.int32),
        ),
    )(x, W_sel, b_sel.reshape(1, NUM_BANKS))


def _sc_scatter_rows_body(src_hbm, pos0_hbm, pos1_hbm, out_hbm, idx_v, rows_v, s0, s1):
    wid = lax.axis_index("s") * 2 + lax.axis_index("c")
    per_w = T // NW
    base = wid * per_w
    h0 = pltpu.async_copy(pos0_hbm.at[pl.ds(base, per_w)], idx_v.at[0], s0)
    h1 = pltpu.async_copy(pos1_hbm.at[pl.ds(base, per_w)], idx_v.at[1], s0)
    h2 = pltpu.async_copy(src_hbm.at[pl.ds(base, per_w)], rows_v, s1)
    h0.wait()
    h1.wait()
    h2.wait()
    c0 = pltpu.async_copy(rows_v, out_hbm.at[idx_v.at[0]], s0)
    c1 = pltpu.async_copy(rows_v, out_hbm.at[idx_v.at[1]], s1)
    c0.wait()
    c1.wait()


def _sc_scatter_rows(src, pos0, pos1):
    per_w = T // NW
    return pl.kernel(
        _sc_scatter_rows_body,
        out_type=jax.ShapeDtypeStruct((ROWS_PAD, D_MODEL), jnp.float32),
        mesh=_sc_mesh(),
        scratch_types=[
            pltpu.VMEM((2, per_w), jnp.int32),
            pltpu.VMEM((per_w, D_MODEL), jnp.float32),
            pltpu.SemaphoreType.DMA,
            pltpu.SemaphoreType.DMA,
        ],
    )(src, pos0, pos1)


NBUF = 3


def _gmm_kernel(
    starts_ref, counts_ref, xs_hbm, w1_hbm, b1_ref, w2_hbm, b2_ref, ys_ref,
    xs_ref, w1b, w2b, semx, sem1, sem2,
):
    e = pl.program_id(0)

    @pl.when(e == 0)
    def _():
        pltpu.make_async_copy(xs_hbm, xs_ref, semx).start()
        for s in range(2):
            pltpu.make_async_copy(w1_hbm.at[s], w1b.at[s], sem1.at[s]).start()
            pltpu.make_async_copy(w2_hbm.at[s], w2b.at[s], sem2.at[s]).start()
        pltpu.make_async_copy(xs_hbm, xs_ref, semx).wait()

    slot = lax.rem(e, NBUF)
    pltpu.make_async_copy(w1_hbm.at[e], w1b.at[slot], sem1.at[slot]).wait()
    pltpu.make_async_copy(w2_hbm.at[e], w2b.at[slot], sem2.at[slot]).wait()

    @pl.when(e + 2 < NUM_BANKS)
    def _():
        nslot = lax.rem(e + 2, NBUF)
        pltpu.make_async_copy(w1_hbm.at[e + 2], w1b.at[nslot], sem1.at[nslot]).start()
        pltpu.make_async_copy(w2_hbm.at[e + 2], w2b.at[nslot], sem2.at[nslot]).start()

    start = pl.multiple_of(starts_ref[e], 8)
    n = counts_ref[e]
    ntiles = (n + TILE_M - 1) // TILE_M
    b1 = b1_ref[0]
    b2 = b2_ref[0]

    def body(j, carry):
        r0 = start + j * TILE_M
        xt = xs_ref[pl.ds(r0, TILE_M), :]
        h = jnp.dot(xt, w1b[slot], preferred_element_type=jnp.float32) + b1
        h = jnp.maximum(h, 0.0)
        yt = jnp.dot(h, w2b[slot], preferred_element_type=jnp.float32) + b2
        ys_ref[pl.ds(r0, TILE_M), :] = yt
        return carry

    lax.fori_loop(0, ntiles, body, 0)


def _gmm(starts, counts, xs, W1, b1, W2, b2):
    f32 = jnp.float32
    return pl.pallas_call(
        _gmm_kernel,
        grid=(NUM_BANKS,),
        in_specs=[
            pl.BlockSpec(memory_space=pltpu.SMEM),
            pl.BlockSpec(memory_space=pltpu.SMEM),
            pl.BlockSpec(memory_space=pl.ANY),
            pl.BlockSpec(memory_space=pl.ANY),
            pl.BlockSpec((1, 1, D_HIDDEN), lambda e: (e, 0, 0)),
            pl.BlockSpec(memory_space=pl.ANY),
            pl.BlockSpec((1, 1, D_MODEL), lambda e: (e, 0, 0)),
        ],
        out_specs=pl.BlockSpec((ROWS_PAD, D_MODEL), lambda e: (0, 0)),
        out_shape=jax.ShapeDtypeStruct((ROWS_PAD, D_MODEL), f32),
        scratch_shapes=[
            pltpu.VMEM((ROWS_PAD, D_MODEL), f32),
            pltpu.VMEM((NBUF, D_MODEL, D_HIDDEN), f32),
            pltpu.VMEM((NBUF, D_HIDDEN, D_MODEL), f32),
            pltpu.SemaphoreType.DMA,
            pltpu.SemaphoreType.DMA((NBUF,)),
            pltpu.SemaphoreType.DMA((NBUF,)),
        ],
    )(starts, counts, xs, W1, b1.reshape(NUM_BANKS, 1, D_HIDDEN), W2, b2.reshape(NUM_BANKS, 1, D_MODEL))


def _sc_combine_body(
    ys_hbm, pos0_hbm, pos1_hbm, p0_hbm, p1_hbm, out_hbm,
    i0_v, i1_v, p0_v, p1_v, r0_v, r1_v, s0, s1, s2, s3, so,
):
    wid = lax.axis_index("s") * 2 + lax.axis_index("c")
    per_w = T // NW
    base = wid * per_w
    ch = per_w // 2
    h0 = pltpu.async_copy(pos0_hbm.at[pl.ds(base, per_w)], i0_v, s0)
    h1 = pltpu.async_copy(pos1_hbm.at[pl.ds(base, per_w)], i1_v, s1)
    h0.wait()
    h1.wait()
    gats = []
    for k, (sa, sb) in enumerate(((s0, s1), (s2, s3))):
        gats.append((
            pltpu.async_copy(
                ys_hbm.at[i0_v.at[pl.ds(k * ch, ch)]], r0_v.at[pl.ds(k * ch, ch)], sa
            ),
            pltpu.async_copy(
                ys_hbm.at[i1_v.at[pl.ds(k * ch, ch)]], r1_v.at[pl.ds(k * ch, ch)], sb
            ),
        ))
    hp0 = pltpu.async_copy(p0_hbm.at[pl.ds(base * 16, per_w * 16)], p0_v, so)
    hp1 = pltpu.async_copy(p1_hbm.at[pl.ds(base * 16, per_w * 16)], p1_v, so)
    hp0.wait()
    hp1.wait()

    def row(r, carry):
        pa = p0_v[pl.ds(r * 16, 16)]
        pb = p1_v[pl.ds(r * 16, 16)]
        for c in range(D_MODEL // 16):
            a = r0_v[r, pl.ds(c * 16, 16)]
            b = r1_v[r, pl.ds(c * 16, 16)]
            r0_v[r, pl.ds(c * 16, 16)] = pa * a + pb * b
        return carry

    outs = []
    for k, (c0, c1) in enumerate(gats):
        c0.wait()
        c1.wait()
        lax.fori_loop(k * ch, (k + 1) * ch, row, 0)
        outs.append(
            pltpu.async_copy(
                r0_v.at[pl.ds(k * ch, ch)], out_hbm.at[pl.ds(base + k * ch, ch)], so
            )
        )
    for o in outs:
        o.wait()


def _sc_combine(ys, pos0, pos1, p0, p1):
    per_w = T // NW
    f32 = jnp.float32
    return pl.kernel(
        _sc_combine_body,
        out_type=jax.ShapeDtypeStruct((T, D_MODEL), f32),
        mesh=_sc_mesh(),
        scratch_types=[
            pltpu.VMEM((per_w,), jnp.int32),
            pltpu.VMEM((per_w,), jnp.int32),
            pltpu.VMEM((per_w * 16,), f32),
            pltpu.VMEM((per_w * 16,), f32),
            pltpu.VMEM((per_w, D_MODEL), f32),
            pltpu.VMEM((per_w, D_MODEL), f32),
            pltpu.SemaphoreType.DMA,
            pltpu.SemaphoreType.DMA,
            pltpu.SemaphoreType.DMA,
            pltpu.SemaphoreType.DMA,
            pltpu.SemaphoreType.DMA,
        ],
    )(ys, pos0, pos1, p0, p1)


def kernel(tensor, W_sel, b_sel, W1, b1, W2, b2):
    x = tensor.reshape(T, D_MODEL)
    p0, p1, pos0, pos1, starts, counts = _selector(x, W_sel, b_sel)
    pos0 = pos0.reshape(T)
    pos1 = pos1.reshape(T)

    xs = _sc_scatter_rows(x, pos0, pos1)
    ys = _gmm(
        starts.reshape(NUM_BANKS), counts.reshape(NUM_BANKS), xs, W1, b1, W2, b2
    )
    out = _sc_combine(ys, pos0, pos1, p0.reshape(T * 16), p1.reshape(T * 16))
    return out.reshape(tensor.shape)
